# baseline (speedup 1.0000x reference)
import functools

import jax
import jax.numpy as jnp
from jax.experimental import pallas as pl
from jax.experimental.pallas import tpu as pltpu


def _rmoe_body(eids_ref, x_ref, w0_ref, w1_ref, b0_ref, b1_ref, o_ref, ws_ref):
    i = pl.program_id(1)

    @pl.when(i == 0)
    def _sum_weights():
        ws_ref[...] = (w0_ref[0] + w1_ref[0]).astype(jnp.bfloat16)

    acc = jax.lax.dot_general(
        x_ref[...].astype(jnp.bfloat16), ws_ref[...],
        dimension_numbers=(((1,), (1,)), ((), ())),
        preferred_element_type=jnp.float32)
    o_ref[...] = acc + (b0_ref[0, 0] + b1_ref[0, 0])[None, :]


@functools.partial(jax.jit, static_argnames=("bm", "bn"))
def _rmoe(x, W, b, expert_ids, bm, bn):
    B, D = x.shape
    nj = D // bn
    ni = B // bm
    grid = (nj, ni)
    eids = expert_ids.astype(jnp.int32)
    b3 = b.reshape(b.shape[0], 1, b.shape[1])

    grid_spec = pltpu.PrefetchScalarGridSpec(
        num_scalar_prefetch=1,
        grid=grid,
        in_specs=[
            pl.BlockSpec((bm, D), lambda j, i, eids: (i, 0)),
            pl.BlockSpec((1, bn, D), lambda j, i, eids: (eids[0], j, 0)),
            pl.BlockSpec((1, bn, D), lambda j, i, eids: (eids[1], j, 0)),
            pl.BlockSpec((1, 1, bn), lambda j, i, eids: (eids[0], 0, j)),
            pl.BlockSpec((1, 1, bn), lambda j, i, eids: (eids[1], 0, j)),
        ],
        out_specs=pl.BlockSpec((bm, bn), lambda j, i, eids: (i, j)),
        scratch_shapes=[pltpu.VMEM((bn, D), jnp.bfloat16)],
    )
    return pl.pallas_call(
        _rmoe_body,
        grid_spec=grid_spec,
        out_shape=jax.ShapeDtypeStruct((B, D), jnp.float32),
        compiler_params=pltpu.CompilerParams(
            dimension_semantics=("arbitrary", "arbitrary")),
    )(eids, x, W, W, b3, b3)


def kernel(x, W, b, expert_ids):
    return _rmoe(x, W, b, expert_ids, bm=1024, bn=512)


# speedup vs baseline: 1.3789x; 1.3789x over previous
import functools

import jax
import jax.numpy as jnp
from jax.experimental import pallas as pl
from jax.experimental.pallas import tpu as pltpu


def _prep_body(eids_ref, w0_ref, w1_ref, ws_ref):
    ws_ref[...] = (w0_ref[0] + w1_ref[0]).astype(jnp.bfloat16)


def _mm_body(eids_ref, x_ref, ws_ref, b0_ref, b1_ref, o_ref):
    acc = jax.lax.dot_general(
        x_ref[...].astype(jnp.bfloat16), ws_ref[...],
        dimension_numbers=(((1,), (1,)), ((), ())),
        preferred_element_type=jnp.float32)
    o_ref[...] = acc + (b0_ref[0, 0] + b1_ref[0, 0])[None, :]


@functools.partial(jax.jit, static_argnames=("bm", "bp"))
def _rmoe(x, W, b, expert_ids, bm, bp):
    B, D = x.shape
    eids = expert_ids.astype(jnp.int32)
    b3 = b.reshape(b.shape[0], 1, b.shape[1])

    prep_spec = pltpu.PrefetchScalarGridSpec(
        num_scalar_prefetch=1,
        grid=(D // bp,),
        in_specs=[
            pl.BlockSpec((1, bp, D), lambda j, eids: (eids[0], j, 0)),
            pl.BlockSpec((1, bp, D), lambda j, eids: (eids[1], j, 0)),
        ],
        out_specs=pl.BlockSpec((bp, D), lambda j, eids: (j, 0)),
    )
    ws = pl.pallas_call(
        _prep_body,
        grid_spec=prep_spec,
        out_shape=jax.ShapeDtypeStruct((D, D), jnp.bfloat16),
    )(eids, W, W)

    mm_spec = pltpu.PrefetchScalarGridSpec(
        num_scalar_prefetch=1,
        grid=(B // bm,),
        in_specs=[
            pl.BlockSpec((bm, D), lambda i, eids: (i, 0)),
            pl.BlockSpec((D, D), lambda i, eids: (0, 0)),
            pl.BlockSpec((1, 1, D), lambda i, eids: (eids[0], 0, 0)),
            pl.BlockSpec((1, 1, D), lambda i, eids: (eids[1], 0, 0)),
        ],
        out_specs=pl.BlockSpec((bm, D), lambda i, eids: (i, 0)),
    )
    return pl.pallas_call(
        _mm_body,
        grid_spec=mm_spec,
        out_shape=jax.ShapeDtypeStruct((B, D), jnp.float32),
        compiler_params=pltpu.CompilerParams(
            dimension_semantics=("arbitrary",)),
    )(eids, x, ws, b3, b3)


def kernel(x, W, b, expert_ids):
    return _rmoe(x, W, b, expert_ids, bm=512, bp=1024)


# speedup vs baseline: 1.4112x; 1.0234x over previous
import functools

import jax
import jax.numpy as jnp
from jax.experimental import pallas as pl
from jax.experimental.pallas import tpu as pltpu


def _prep_body(eids_ref, w0_ref, w1_ref, ws_ref):
    ws_ref[...] = (w0_ref[0] + w1_ref[0]).astype(jnp.bfloat16)


def _mm_body(eids_ref, x_ref, ws_ref, b0_ref, b1_ref, o_ref):
    acc = jax.lax.dot_general(
        x_ref[...].astype(jnp.bfloat16), ws_ref[...],
        dimension_numbers=(((1,), (1,)), ((), ())),
        preferred_element_type=jnp.float32)
    o_ref[...] = acc + (b0_ref[0, 0] + b1_ref[0, 0])[None, :]


@functools.partial(jax.jit, static_argnames=("bm", "bp"))
def _rmoe(x, W, b, expert_ids, bm, bp):
    B, D = x.shape
    eids = expert_ids.astype(jnp.int32)
    b3 = b.reshape(b.shape[0], 1, b.shape[1])

    prep_spec = pltpu.PrefetchScalarGridSpec(
        num_scalar_prefetch=1,
        grid=(D // bp,),
        in_specs=[
            pl.BlockSpec((1, bp, D), lambda j, eids: (eids[0], j, 0)),
            pl.BlockSpec((1, bp, D), lambda j, eids: (eids[1], j, 0)),
        ],
        out_specs=pl.BlockSpec((bp, D), lambda j, eids: (j, 0)),
    )
    ws = pl.pallas_call(
        _prep_body,
        grid_spec=prep_spec,
        out_shape=jax.ShapeDtypeStruct((D, D), jnp.bfloat16),
    )(eids, W, W)

    mm_spec = pltpu.PrefetchScalarGridSpec(
        num_scalar_prefetch=1,
        grid=(B // bm,),
        in_specs=[
            pl.BlockSpec((bm, D), lambda i, eids: (i, 0)),
            pl.BlockSpec((D, D), lambda i, eids: (0, 0)),
            pl.BlockSpec((1, 1, D), lambda i, eids: (eids[0], 0, 0)),
            pl.BlockSpec((1, 1, D), lambda i, eids: (eids[1], 0, 0)),
        ],
        out_specs=pl.BlockSpec((bm, D), lambda i, eids: (i, 0)),
    )
    return pl.pallas_call(
        _mm_body,
        grid_spec=mm_spec,
        out_shape=jax.ShapeDtypeStruct((B, D), jnp.float32),
        compiler_params=pltpu.CompilerParams(
            dimension_semantics=("arbitrary",)),
    )(eids, x, ws, b3, b3)


def kernel(x, W, b, expert_ids):
    return _rmoe(x, W, b, expert_ids, bm=1024, bp=1024)


# speedup vs baseline: 1.4197x; 1.0061x over previous
import functools

import jax
import jax.numpy as jnp
from jax.experimental import pallas as pl
from jax.experimental.pallas import tpu as pltpu


def _prep_body(eids_ref, w0_ref, w1_ref, ws_ref):
    ws_ref[...] = (w0_ref[0] + w1_ref[0]).astype(jnp.bfloat16).T


def _mm_body(eids_ref, x_ref, ws_ref, b0_ref, b1_ref, o_ref):
    acc = jax.lax.dot_general(
        x_ref[...].astype(jnp.bfloat16), ws_ref[...],
        dimension_numbers=(((1,), (0,)), ((), ())),
        preferred_element_type=jnp.float32)
    o_ref[...] = acc + (b0_ref[0, 0] + b1_ref[0, 0])[None, :]


@functools.partial(jax.jit, static_argnames=("bm", "bp"))
def _rmoe(x, W, b, expert_ids, bm, bp):
    B, D = x.shape
    eids = expert_ids.astype(jnp.int32)
    b3 = b.reshape(b.shape[0], 1, b.shape[1])

    prep_spec = pltpu.PrefetchScalarGridSpec(
        num_scalar_prefetch=1,
        grid=(D // bp,),
        in_specs=[
            pl.BlockSpec((1, bp, D), lambda j, eids: (eids[0], j, 0)),
            pl.BlockSpec((1, bp, D), lambda j, eids: (eids[1], j, 0)),
        ],
        out_specs=pl.BlockSpec((D, bp), lambda j, eids: (0, j)),
    )
    ws = pl.pallas_call(
        _prep_body,
        grid_spec=prep_spec,
        out_shape=jax.ShapeDtypeStruct((D, D), jnp.bfloat16),
    )(eids, W, W)

    mm_spec = pltpu.PrefetchScalarGridSpec(
        num_scalar_prefetch=1,
        grid=(B // bm,),
        in_specs=[
            pl.BlockSpec((bm, D), lambda i, eids: (i, 0)),
            pl.BlockSpec((D, D), lambda i, eids: (0, 0)),
            pl.BlockSpec((1, 1, D), lambda i, eids: (eids[0], 0, 0)),
            pl.BlockSpec((1, 1, D), lambda i, eids: (eids[1], 0, 0)),
        ],
        out_specs=pl.BlockSpec((bm, D), lambda i, eids: (i, 0)),
    )
    return pl.pallas_call(
        _mm_body,
        grid_spec=mm_spec,
        out_shape=jax.ShapeDtypeStruct((B, D), jnp.float32),
        compiler_params=pltpu.CompilerParams(
            dimension_semantics=("arbitrary",)),
    )(eids, x, ws, b3, b3)


def kernel(x, W, b, expert_ids):
    return _rmoe(x, W, b, expert_ids, bm=1024, bp=1024)
